# baseline (speedup 1.0000x reference)
import functools

import jax
import jax.numpy as jnp
from jax import lax
from jax.experimental import pallas as pl
from jax.experimental.pallas import tpu as pltpu
from jax.experimental.pallas import tpu_sc as plsc

_NC = 2
_NS = 16
_IN_F = 6
_OUT_F = 8
_SB = 64


def _sc_linear(values, wb, T):
    n_workers = _NC * _NS
    chunk = T // n_workers

    mesh = plsc.VectorSubcoreMesh(
        core_axis_name="c", subcore_axis_name="s",
        num_cores=_NC, num_subcores=_NS)

    @functools.partial(
        pl.kernel,
        out_type=jax.ShapeDtypeStruct((T, _OUT_F), jnp.float32),
        mesh=mesh,
        scratch_types=[
            pltpu.VMEM((chunk, _IN_F), jnp.float32),
            pltpu.VMEM((chunk, _OUT_F), jnp.float32),
            pltpu.VMEM((64,), jnp.float32),
        ],
        compiler_params=pltpu.CompilerParams(
            needs_layout_passes=False,
            use_tc_tiling_on_sc=False,
        ),
    )
    def run(v_hbm, wb_hbm, out_hbm, v_vmem, o_vmem, wb_vmem):
        wid = lax.axis_index("s") * _NC + lax.axis_index("c")
        base = wid * chunk
        pltpu.sync_copy(wb_hbm, wb_vmem)
        pltpu.sync_copy(v_hbm.at[pl.ds(base, chunk), :], v_vmem)

        wv = [wb_vmem[pl.ds(16 * j, 16)] for j in range(4)]

        def splat(j):
            idx = jnp.full((16,), j % 16, jnp.int32)
            return wv[j // 16].at[idx].get(mode="promise_in_bounds")

        lane = lax.iota(jnp.int32, 16)
        groups = _SB // 16
        icols = [jnp.full((16,), i, jnp.int32) for i in range(_IN_F)]
        ocols = [jnp.full((16,), o, jnp.int32) for o in range(_OUT_F)]

        def step(k, carry):
            t0 = k * _SB
            rows = [lane + (t0 + 16 * g) for g in range(groups)]
            vi = [[plsc.load_gather(v_vmem, [rows[g], icols[i]])
                   for i in range(_IN_F)] for g in range(groups)]
            for o in range(_OUT_F):
                wo = [splat(o * _IN_F + i) for i in range(_IN_F)]
                bo = splat(48 + o)
                for g in range(groups):
                    acc = bo
                    for i in range(_IN_F):
                        acc = acc + vi[g][i] * wo[i]
                    plsc.store_scatter(o_vmem, [rows[g], ocols[o]], acc)
            return carry

        lax.fori_loop(0, chunk // _SB, step, 0)
        pltpu.sync_copy(o_vmem, out_hbm.at[pl.ds(base, chunk), :])

    return run(values, wb)


def kernel(values, offsets, W, b):
    del offsets
    T = values.shape[0]
    wb = jnp.pad(jnp.concatenate([W.reshape(-1), b]), (0, 8))
    return _sc_linear(values, wb, T)


# speedup vs baseline: 3.3282x; 3.3282x over previous
import functools

import jax
import jax.numpy as jnp
from jax import lax
from jax.experimental import pallas as pl
from jax.experimental.pallas import tpu as pltpu
from jax.experimental.pallas import tpu_sc as plsc

_NC = 2
_NS = 16
_IN_F = 6
_OUT_F = 8
_SB = 64


def _sc_linear(v_t, wb, T):
    n_workers = _NC * _NS
    chunk = T // n_workers

    mesh = plsc.VectorSubcoreMesh(
        core_axis_name="c", subcore_axis_name="s",
        num_cores=_NC, num_subcores=_NS)

    @functools.partial(
        pl.kernel,
        out_type=jax.ShapeDtypeStruct((_OUT_F, T), jnp.float32),
        mesh=mesh,
        scratch_types=[
            pltpu.VMEM((_IN_F, chunk), jnp.float32),
            pltpu.VMEM((_OUT_F, chunk), jnp.float32),
            pltpu.VMEM((64,), jnp.float32),
        ],
        compiler_params=pltpu.CompilerParams(needs_layout_passes=False),
    )
    def run(v_hbm, wb_hbm, out_hbm, v_vmem, o_vmem, wb_vmem):
        wid = lax.axis_index("s") * _NC + lax.axis_index("c")
        base = wid * chunk
        pltpu.sync_copy(wb_hbm, wb_vmem)
        pltpu.sync_copy(v_hbm.at[:, pl.ds(base, chunk)], v_vmem)

        wv = [wb_vmem[pl.ds(16 * j, 16)] for j in range(4)]

        def splat(j):
            idx = jnp.full((16,), j % 16, jnp.int32)
            return wv[j // 16].at[idx].get(mode="promise_in_bounds")

        groups = _SB // 16

        def step(k, carry):
            t0 = k * _SB
            vi = [[v_vmem[i, pl.ds(t0 + 16 * g, 16)]
                   for i in range(_IN_F)] for g in range(groups)]
            for o in range(_OUT_F):
                wo = [splat(o * _IN_F + i) for i in range(_IN_F)]
                bo = splat(48 + o)
                for g in range(groups):
                    acc = bo
                    for i in range(_IN_F):
                        acc = acc + vi[g][i] * wo[i]
                    o_vmem[o, pl.ds(t0 + 16 * g, 16)] = acc
            return carry

        lax.fori_loop(0, chunk // _SB, step, 0)
        pltpu.sync_copy(o_vmem, out_hbm.at[:, pl.ds(base, chunk)])

    return run(v_t, wb)


def kernel(values, offsets, W, b):
    del offsets
    T = values.shape[0]
    wb = jnp.pad(jnp.concatenate([W.reshape(-1), b]), (0, 8))
    out_t = _sc_linear(values.T, wb, T)
    return out_t.T
